# baseline (speedup 1.0000x reference)
import functools

import jax
import jax.numpy as jnp
from jax import lax
from jax.experimental import pallas as pl
from jax.experimental.pallas import tpu as pltpu
from jax.experimental.pallas import tpu_sc as plsc

N_NODES = 10000
N_EDGES = 320000
D_FEAT = 128

_info = plsc.get_sparse_core_info()
_NC, _NS, _L = _info.num_cores, _info.num_subcores, _info.num_lanes
_NW = _NC * _NS
_EPW = N_EDGES // _NW
_UNROLL = 8
_AWIN = ((_EPW + 127) // 128 + 1) * 128 - 128 + 128


def _proj_body(emb_ref, w_ref, b_ref, u_ref, v_ref):
    e = emb_ref[...]
    wd = w_ref[1:2, :, :] - w_ref[0:1, :, :]
    wu = wd[0, 0:1, :]
    wv = wd[0, 1:2, :]
    db = b_ref[0, 1] - b_ref[0, 0]
    dn = (((1,), (1,)), ((), ()))
    qu = lax.dot_general(wu, e, dn, preferred_element_type=jnp.float32)
    qv = lax.dot_general(wv, e, dn, preferred_element_type=jnp.float32)
    u_ref[...] = jnp.exp(jnp.clip(qu[0, :] + db, -60.0, 60.0))
    v_ref[...] = jnp.exp(jnp.clip(qv[0, :], -60.0, 60.0))


def _project(emb, W, b):
    return pl.pallas_call(
        _proj_body,
        out_shape=[
            jax.ShapeDtypeStruct((N_NODES,), jnp.float32),
            jax.ShapeDtypeStruct((N_NODES,), jnp.float32),
        ],
        in_specs=[
            pl.BlockSpec(memory_space=pltpu.VMEM),
            pl.BlockSpec(memory_space=pltpu.VMEM),
            pl.BlockSpec(memory_space=pltpu.SMEM),
        ],
        out_specs=[
            pl.BlockSpec(memory_space=pltpu.VMEM),
            pl.BlockSpec(memory_space=pltpu.VMEM),
        ],
    )(emb, W.reshape(2, 2, D_FEAT), b.reshape(1, 2))


_mesh = plsc.VectorSubcoreMesh(core_axis_name="c", subcore_axis_name="s")


@functools.partial(
    pl.kernel,
    mesh=_mesh,
    compiler_params=pltpu.CompilerParams(needs_layout_passes=False),
    out_type=jax.ShapeDtypeStruct((2 * N_EDGES,), jnp.float32),
    scratch_types=[
        pltpu.VMEM((N_NODES,), jnp.float32),
        pltpu.VMEM((N_NODES,), jnp.float32),
        pltpu.VMEM((_AWIN,), jnp.int32),
        pltpu.VMEM((_EPW,), jnp.float32),
        pltpu.VMEM((_EPW,), jnp.float32),
        pltpu.SemaphoreType.DMA,
    ],
)
def _edge_kernel(u_hbm, v_hbm, pk_hbm, out_hbm,
                 u_v, v_v, idx_v, p0_v, p1_v, sem):
    wid = lax.axis_index("s") * _NC + lax.axis_index("c")
    base = wid * _EPW
    lo = pl.multiple_of((base // 128) * 128, 128)
    sh = base - lo
    c1 = pltpu.async_copy(u_hbm, u_v, sem)
    c2 = pltpu.async_copy(v_hbm, v_v, sem)
    c3 = pltpu.async_copy(pk_hbm.at[pl.ds(lo, _AWIN)], idx_v, sem)
    c1.wait()
    c2.wait()
    c3.wait()

    half = _EPW // 2

    @plsc.parallel_loop(0, half // _L, step=1, unroll=_UNROLL)
    def _loop_a(i):
        off = i * _L
        pk = idx_v[pl.ds(sh + off, _L)]
        r = pk & 0xFFFF
        c = lax.shift_right_logical(pk, 16)
        us = plsc.load_gather(u_v, [r])
        vs = plsc.load_gather(v_v, [c])
        ed = us * vs
        p0 = 1.0 / (1.0 + ed)
        p0_v[pl.ds(off, _L)] = p0
        p1_v[pl.ds(off, _L)] = 1.0 - p0
    o1 = pltpu.async_copy(p0_v.at[pl.ds(0, half)], out_hbm.at[pl.ds(base, half)], sem)
    o2 = pltpu.async_copy(p1_v.at[pl.ds(0, half)],
                          out_hbm.at[pl.ds(N_EDGES + base, half)], sem)

    @plsc.parallel_loop(half // _L, _EPW // _L, step=1, unroll=_UNROLL)
    def _loop_b(i):
        off = i * _L
        pk = idx_v[pl.ds(sh + off, _L)]
        r = pk & 0xFFFF
        c = lax.shift_right_logical(pk, 16)
        us = plsc.load_gather(u_v, [r])
        vs = plsc.load_gather(v_v, [c])
        ed = us * vs
        p0 = 1.0 / (1.0 + ed)
        p0_v[pl.ds(off, _L)] = p0
        p1_v[pl.ds(off, _L)] = 1.0 - p0
    o3 = pltpu.async_copy(p0_v.at[pl.ds(half, half)],
                          out_hbm.at[pl.ds(base + half, half)], sem)
    o4 = pltpu.async_copy(p1_v.at[pl.ds(half, half)],
                          out_hbm.at[pl.ds(N_EDGES + base + half, half)], sem)
    o1.wait()
    o2.wait()
    o3.wait()
    o4.wait()


def kernel(node_embeddings, edge_index, W, b):
    u, v = _project(node_embeddings, W, b)
    eidx = edge_index.astype(jnp.int32)
    pk = eidx[0] | (eidx[1] << 16)
    out_flat = _edge_kernel(u, v, pk)
    return out_flat.reshape(2, N_EDGES).T


# speedup vs baseline: 1.3854x; 1.3854x over previous
import functools

import jax
import jax.numpy as jnp
from jax import lax
from jax.experimental import pallas as pl
from jax.experimental.pallas import tpu as pltpu
from jax.experimental.pallas import tpu_sc as plsc

N_NODES = 10000
N_EDGES = 320000
D_FEAT = 128

_info = plsc.get_sparse_core_info()
_NC, _NS, _L = _info.num_cores, _info.num_subcores, _info.num_lanes
_NW = _NC * _NS
_EPW = N_EDGES // _NW
_UNROLL = 16
_AWIN = ((_EPW + 127) // 128 + 1) * 128 - 128 + 128


def _proj_body(emb_ref, w_ref, b_ref, u_ref, v_ref):
    e = emb_ref[...]
    wd = w_ref[1:2, :, :] - w_ref[0:1, :, :]
    wu = wd[0, 0:1, :]
    wv = wd[0, 1:2, :]
    db = b_ref[0, 1] - b_ref[0, 0]
    dn = (((1,), (1,)), ((), ()))
    qu = lax.dot_general(wu, e, dn, preferred_element_type=jnp.float32)
    qv = lax.dot_general(wv, e, dn, preferred_element_type=jnp.float32)
    u_ref[...] = jnp.exp(jnp.clip(qu[0, :] + db, -60.0, 60.0))
    v_ref[...] = jnp.exp(jnp.clip(qv[0, :], -60.0, 60.0))


def _project(emb, W, b):
    return pl.pallas_call(
        _proj_body,
        out_shape=[
            jax.ShapeDtypeStruct((N_NODES,), jnp.float32),
            jax.ShapeDtypeStruct((N_NODES,), jnp.float32),
        ],
        in_specs=[
            pl.BlockSpec(memory_space=pltpu.VMEM),
            pl.BlockSpec(memory_space=pltpu.VMEM),
            pl.BlockSpec(memory_space=pltpu.SMEM),
        ],
        out_specs=[
            pl.BlockSpec(memory_space=pltpu.VMEM),
            pl.BlockSpec(memory_space=pltpu.VMEM),
        ],
    )(emb, W.reshape(2, 2, D_FEAT), b.reshape(1, 2))


_mesh = plsc.VectorSubcoreMesh(core_axis_name="c", subcore_axis_name="s")


@functools.partial(
    pl.kernel,
    mesh=_mesh,
    compiler_params=pltpu.CompilerParams(needs_layout_passes=False),
    out_type=jax.ShapeDtypeStruct((2 * N_EDGES,), jnp.float32),
    scratch_types=[
        pltpu.VMEM((N_NODES,), jnp.float32),
        pltpu.VMEM((N_NODES,), jnp.float32),
        pltpu.VMEM((2, _AWIN), jnp.int32),
        pltpu.VMEM((_EPW,), jnp.float32),
        pltpu.VMEM((_EPW,), jnp.float32),
        pltpu.SemaphoreType.DMA,
    ],
)
def _edge_kernel(u_hbm, v_hbm, eidx_hbm, out_hbm,
                 u_v, v_v, idx_v, p0_v, p1_v, sem):
    wid = lax.axis_index("s") * _NC + lax.axis_index("c")
    base = wid * _EPW
    lo = pl.multiple_of((base // 128) * 128, 128)
    sh = base - lo
    c1 = pltpu.async_copy(u_hbm, u_v, sem)
    c2 = pltpu.async_copy(v_hbm, v_v, sem)
    c3 = pltpu.async_copy(eidx_hbm.at[:, pl.ds(lo, _AWIN)], idx_v, sem)
    c1.wait()
    c2.wait()
    c3.wait()

    half = _EPW // 2

    @plsc.parallel_loop(0, half // _L, step=1, unroll=_UNROLL)
    def _loop_a(i):
        off = i * _L
        r = idx_v[0, pl.ds(sh + off, _L)]
        c = idx_v[1, pl.ds(sh + off, _L)]
        us = plsc.load_gather(u_v, [r])
        vs = plsc.load_gather(v_v, [c])
        ed = us * vs
        p0 = 1.0 / (1.0 + ed)
        p0_v[pl.ds(off, _L)] = p0
        p1_v[pl.ds(off, _L)] = 1.0 - p0
    o1 = pltpu.async_copy(p0_v.at[pl.ds(0, half)], out_hbm.at[pl.ds(base, half)], sem)
    o2 = pltpu.async_copy(p1_v.at[pl.ds(0, half)],
                          out_hbm.at[pl.ds(N_EDGES + base, half)], sem)

    @plsc.parallel_loop(half // _L, _EPW // _L, step=1, unroll=_UNROLL)
    def _loop_b(i):
        off = i * _L
        r = idx_v[0, pl.ds(sh + off, _L)]
        c = idx_v[1, pl.ds(sh + off, _L)]
        us = plsc.load_gather(u_v, [r])
        vs = plsc.load_gather(v_v, [c])
        ed = us * vs
        p0 = 1.0 / (1.0 + ed)
        p0_v[pl.ds(off, _L)] = p0
        p1_v[pl.ds(off, _L)] = 1.0 - p0
    o3 = pltpu.async_copy(p0_v.at[pl.ds(half, half)],
                          out_hbm.at[pl.ds(base + half, half)], sem)
    o4 = pltpu.async_copy(p1_v.at[pl.ds(half, half)],
                          out_hbm.at[pl.ds(N_EDGES + base + half, half)], sem)
    o1.wait()
    o2.wait()
    o3.wait()
    o4.wait()


def kernel(node_embeddings, edge_index, W, b):
    u, v = _project(node_embeddings, W, b)
    eidx = edge_index.astype(jnp.int32)
    out_flat = _edge_kernel(u, v, eidx)
    return out_flat.reshape(2, N_EDGES).T


# speedup vs baseline: 1.4119x; 1.0192x over previous
import functools

import jax
import jax.numpy as jnp
from jax import lax
from jax.experimental import pallas as pl
from jax.experimental.pallas import tpu as pltpu
from jax.experimental.pallas import tpu_sc as plsc

N_NODES = 10000
N_EDGES = 320000
D_FEAT = 128

_info = plsc.get_sparse_core_info()
_NC, _NS, _L = _info.num_cores, _info.num_subcores, _info.num_lanes
_NW = _NC * _NS
_EPW = N_EDGES // _NW
_UNROLL = 4
_AWIN = ((_EPW + 127) // 128 + 1) * 128 - 128 + 128


def _proj_body(emb_ref, w_ref, b_ref, u_ref, v_ref):
    e = emb_ref[...]
    wd = w_ref[1:2, :, :] - w_ref[0:1, :, :]
    wu = wd[0, 0:1, :]
    wv = wd[0, 1:2, :]
    db = b_ref[0, 1] - b_ref[0, 0]
    dn = (((1,), (1,)), ((), ()))
    qu = lax.dot_general(wu, e, dn, preferred_element_type=jnp.float32)
    qv = lax.dot_general(wv, e, dn, preferred_element_type=jnp.float32)
    u_ref[...] = jnp.exp(jnp.clip(qu[0, :] + db, -60.0, 60.0))
    v_ref[...] = jnp.exp(jnp.clip(qv[0, :], -60.0, 60.0))


def _project(emb, W, b):
    return pl.pallas_call(
        _proj_body,
        out_shape=[
            jax.ShapeDtypeStruct((N_NODES,), jnp.float32),
            jax.ShapeDtypeStruct((N_NODES,), jnp.float32),
        ],
        in_specs=[
            pl.BlockSpec(memory_space=pltpu.VMEM),
            pl.BlockSpec(memory_space=pltpu.VMEM),
            pl.BlockSpec(memory_space=pltpu.SMEM),
        ],
        out_specs=[
            pl.BlockSpec(memory_space=pltpu.VMEM),
            pl.BlockSpec(memory_space=pltpu.VMEM),
        ],
    )(emb, W.reshape(2, 2, D_FEAT), b.reshape(1, 2))


_mesh = plsc.VectorSubcoreMesh(core_axis_name="c", subcore_axis_name="s")


@functools.partial(
    pl.kernel,
    mesh=_mesh,
    compiler_params=pltpu.CompilerParams(needs_layout_passes=False),
    out_type=jax.ShapeDtypeStruct((2 * N_EDGES,), jnp.float32),
    scratch_types=[
        pltpu.VMEM((N_NODES,), jnp.float32),
        pltpu.VMEM((N_NODES,), jnp.float32),
        pltpu.VMEM((2, _AWIN), jnp.int32),
        pltpu.VMEM((_EPW,), jnp.float32),
        pltpu.VMEM((_EPW,), jnp.float32),
        pltpu.SemaphoreType.DMA,
    ],
)
def _edge_kernel(u_hbm, v_hbm, eidx_hbm, out_hbm,
                 u_v, v_v, idx_v, p0_v, p1_v, sem):
    wid = lax.axis_index("s") * _NC + lax.axis_index("c")
    base = wid * _EPW
    lo = pl.multiple_of((base // 128) * 128, 128)
    sh = base - lo
    c1 = pltpu.async_copy(u_hbm, u_v, sem)
    c2 = pltpu.async_copy(v_hbm, v_v, sem)
    c3 = pltpu.async_copy(eidx_hbm.at[:, pl.ds(lo, _AWIN)], idx_v, sem)
    c1.wait()
    c2.wait()
    c3.wait()

    half = _EPW // 2

    @plsc.parallel_loop(0, half // _L, step=1, unroll=_UNROLL)
    def _loop_a(i):
        off = i * _L
        r = idx_v[0, pl.ds(sh + off, _L)]
        c = idx_v[1, pl.ds(sh + off, _L)]
        us = plsc.load_gather(u_v, [r])
        vs = plsc.load_gather(v_v, [c])
        ed = us * vs
        p0 = 1.0 / (1.0 + ed)
        p0_v[pl.ds(off, _L)] = p0
        p1_v[pl.ds(off, _L)] = 1.0 - p0
    o1 = pltpu.async_copy(p0_v.at[pl.ds(0, half)], out_hbm.at[pl.ds(base, half)], sem)
    o2 = pltpu.async_copy(p1_v.at[pl.ds(0, half)],
                          out_hbm.at[pl.ds(N_EDGES + base, half)], sem)

    @plsc.parallel_loop(half // _L, _EPW // _L, step=1, unroll=_UNROLL)
    def _loop_b(i):
        off = i * _L
        r = idx_v[0, pl.ds(sh + off, _L)]
        c = idx_v[1, pl.ds(sh + off, _L)]
        us = plsc.load_gather(u_v, [r])
        vs = plsc.load_gather(v_v, [c])
        ed = us * vs
        p0 = 1.0 / (1.0 + ed)
        p0_v[pl.ds(off, _L)] = p0
        p1_v[pl.ds(off, _L)] = 1.0 - p0
    o3 = pltpu.async_copy(p0_v.at[pl.ds(half, half)],
                          out_hbm.at[pl.ds(base + half, half)], sem)
    o4 = pltpu.async_copy(p1_v.at[pl.ds(half, half)],
                          out_hbm.at[pl.ds(N_EDGES + base + half, half)], sem)
    o1.wait()
    o2.wait()
    o3.wait()
    o4.wait()


def kernel(node_embeddings, edge_index, W, b):
    u, v = _project(node_embeddings, W, b)
    eidx = edge_index.astype(jnp.int32)
    out_flat = _edge_kernel(u, v, eidx)
    return out_flat.reshape(2, N_EDGES).T


# speedup vs baseline: 1.5139x; 1.0722x over previous
import functools

import jax
import jax.numpy as jnp
from jax import lax
from jax.experimental import pallas as pl
from jax.experimental.pallas import tpu as pltpu
from jax.experimental.pallas import tpu_sc as plsc

N_NODES = 10000
N_EDGES = 320000
D_FEAT = 128

_info = plsc.get_sparse_core_info()
_NC, _NS, _L = _info.num_cores, _info.num_subcores, _info.num_lanes
_NW = _NC * _NS
_EPW = N_EDGES // _NW
_UNROLL = 8
_AWIN = ((_EPW + 127) // 128 + 1) * 128 - 128 + 128


def _proj_body(emb_ref, w_ref, b_ref, u_ref, v_ref):
    e = emb_ref[...]
    wd = w_ref[1:2, :, :] - w_ref[0:1, :, :]
    wu = wd[0, 0:1, :]
    wv = wd[0, 1:2, :]
    db = b_ref[0, 1] - b_ref[0, 0]
    dn = (((1,), (1,)), ((), ()))
    qu = lax.dot_general(wu, e, dn, preferred_element_type=jnp.float32)
    qv = lax.dot_general(wv, e, dn, preferred_element_type=jnp.float32)
    u_ref[...] = jnp.exp(jnp.clip(qu[0, :] + db, -60.0, 60.0))
    v_ref[...] = jnp.exp(jnp.clip(qv[0, :], -60.0, 60.0))


def _project(emb, W, b):
    return pl.pallas_call(
        _proj_body,
        out_shape=[
            jax.ShapeDtypeStruct((N_NODES,), jnp.float32),
            jax.ShapeDtypeStruct((N_NODES,), jnp.float32),
        ],
        in_specs=[
            pl.BlockSpec(memory_space=pltpu.VMEM),
            pl.BlockSpec(memory_space=pltpu.VMEM),
            pl.BlockSpec(memory_space=pltpu.SMEM),
        ],
        out_specs=[
            pl.BlockSpec(memory_space=pltpu.VMEM),
            pl.BlockSpec(memory_space=pltpu.VMEM),
        ],
    )(emb, W.reshape(2, 2, D_FEAT), b.reshape(1, 2))


_mesh = plsc.VectorSubcoreMesh(core_axis_name="c", subcore_axis_name="s")


@functools.partial(
    pl.kernel,
    mesh=_mesh,
    compiler_params=pltpu.CompilerParams(needs_layout_passes=False),
    out_type=jax.ShapeDtypeStruct((2 * N_EDGES,), jnp.float32),
    scratch_types=[
        pltpu.VMEM((N_NODES,), jnp.float32),
        pltpu.VMEM((N_NODES,), jnp.float32),
        pltpu.VMEM((2, _AWIN), jnp.int32),
        pltpu.VMEM((_EPW,), jnp.float32),
        pltpu.VMEM((_EPW,), jnp.float32),
        pltpu.VMEM_SHARED((N_NODES,), jnp.float32),
        pltpu.VMEM_SHARED((N_NODES,), jnp.float32),
        pltpu.SemaphoreType.DMA,
    ],
)
def _edge_kernel(u_hbm, v_hbm, eidx_hbm, out_hbm,
                 u_v, v_v, idx_v, p0_v, p1_v, u_sh, v_sh, sem):
    sid = lax.axis_index("s")
    wid = sid * _NC + lax.axis_index("c")
    base = wid * _EPW
    lo = pl.multiple_of((base // 128) * 128, 128)
    sh = base - lo
    c3 = pltpu.async_copy(eidx_hbm.at[:, pl.ds(lo, _AWIN)], idx_v, sem)

    @pl.when(sid == 0)
    def _fill():
        pltpu.sync_copy(u_hbm, u_sh)
        pltpu.sync_copy(v_hbm, v_sh)
    plsc.subcore_barrier()
    c1 = pltpu.async_copy(u_sh, u_v, sem)
    c2 = pltpu.async_copy(v_sh, v_v, sem)
    c1.wait()
    c2.wait()
    c3.wait()

    half = _EPW // 2

    @plsc.parallel_loop(0, half // _L, step=1, unroll=_UNROLL)
    def _loop_a(i):
        off = i * _L
        r = idx_v[0, pl.ds(sh + off, _L)]
        c = idx_v[1, pl.ds(sh + off, _L)]
        us = plsc.load_gather(u_v, [r])
        vs = plsc.load_gather(v_v, [c])
        ed = us * vs
        p0 = 1.0 / (1.0 + ed)
        p0_v[pl.ds(off, _L)] = p0
        p1_v[pl.ds(off, _L)] = 1.0 - p0
    o1 = pltpu.async_copy(p0_v.at[pl.ds(0, half)], out_hbm.at[pl.ds(base, half)], sem)
    o2 = pltpu.async_copy(p1_v.at[pl.ds(0, half)],
                          out_hbm.at[pl.ds(N_EDGES + base, half)], sem)

    @plsc.parallel_loop(half // _L, _EPW // _L, step=1, unroll=_UNROLL)
    def _loop_b(i):
        off = i * _L
        r = idx_v[0, pl.ds(sh + off, _L)]
        c = idx_v[1, pl.ds(sh + off, _L)]
        us = plsc.load_gather(u_v, [r])
        vs = plsc.load_gather(v_v, [c])
        ed = us * vs
        p0 = 1.0 / (1.0 + ed)
        p0_v[pl.ds(off, _L)] = p0
        p1_v[pl.ds(off, _L)] = 1.0 - p0
    o3 = pltpu.async_copy(p0_v.at[pl.ds(half, half)],
                          out_hbm.at[pl.ds(base + half, half)], sem)
    o4 = pltpu.async_copy(p1_v.at[pl.ds(half, half)],
                          out_hbm.at[pl.ds(N_EDGES + base + half, half)], sem)
    o1.wait()
    o2.wait()
    o3.wait()
    o4.wait()


def kernel(node_embeddings, edge_index, W, b):
    u, v = _project(node_embeddings, W, b)
    eidx = edge_index.astype(jnp.int32)
    out_flat = _edge_kernel(u, v, eidx)
    return out_flat.reshape(2, N_EDGES).T


# speedup vs baseline: 1.5286x; 1.0097x over previous
import functools

import jax
import jax.numpy as jnp
from jax import lax
from jax.experimental import pallas as pl
from jax.experimental.pallas import tpu as pltpu
from jax.experimental.pallas import tpu_sc as plsc

N_NODES = 10000
N_EDGES = 320000
D_FEAT = 128

_info = plsc.get_sparse_core_info()
_NC, _NS, _L = _info.num_cores, _info.num_subcores, _info.num_lanes
_NW = _NC * _NS
_EPW = N_EDGES // _NW
_UNROLL = 8
_AWIN = ((_EPW + 127) // 128 + 1) * 128 - 128 + 128
_AW2 = 5120


def _proj_body(emb_ref, w_ref, b_ref, u_ref, v_ref):
    e = emb_ref[...]
    wd = w_ref[1:2, :, :] - w_ref[0:1, :, :]
    wu = wd[0, 0:1, :]
    wv = wd[0, 1:2, :]
    db = b_ref[0, 1] - b_ref[0, 0]
    dn = (((1,), (1,)), ((), ()))
    qu = lax.dot_general(wu, e, dn, preferred_element_type=jnp.float32)
    qv = lax.dot_general(wv, e, dn, preferred_element_type=jnp.float32)
    u_ref[...] = jnp.exp(jnp.clip(qu[0, :] + db, -60.0, 60.0))
    v_ref[...] = jnp.exp(jnp.clip(qv[0, :], -60.0, 60.0))


def _project(emb, W, b):
    return pl.pallas_call(
        _proj_body,
        out_shape=[
            jax.ShapeDtypeStruct((N_NODES,), jnp.float32),
            jax.ShapeDtypeStruct((N_NODES,), jnp.float32),
        ],
        in_specs=[
            pl.BlockSpec(memory_space=pltpu.VMEM),
            pl.BlockSpec(memory_space=pltpu.VMEM),
            pl.BlockSpec(memory_space=pltpu.SMEM),
        ],
        out_specs=[
            pl.BlockSpec(memory_space=pltpu.VMEM),
            pl.BlockSpec(memory_space=pltpu.VMEM),
        ],
    )(emb, W.reshape(2, 2, D_FEAT), b.reshape(1, 2))


_mesh = plsc.VectorSubcoreMesh(core_axis_name="c", subcore_axis_name="s")


@functools.partial(
    pl.kernel,
    mesh=_mesh,
    compiler_params=pltpu.CompilerParams(needs_layout_passes=False),
    out_type=jax.ShapeDtypeStruct((2 * N_EDGES,), jnp.float32),
    scratch_types=[
        pltpu.VMEM((N_NODES,), jnp.float32),
        pltpu.VMEM((N_NODES,), jnp.float32),
        pltpu.VMEM((2, _AWIN), jnp.int32),
        pltpu.VMEM((_EPW,), jnp.float32),
        pltpu.VMEM((_EPW,), jnp.float32),
        pltpu.VMEM_SHARED((N_NODES,), jnp.float32),
        pltpu.VMEM_SHARED((N_NODES,), jnp.float32),
        pltpu.SemaphoreType.DMA,
    ],
)
def _edge_kernel(u_hbm, v_hbm, eidx_hbm, out_hbm,
                 u_v, v_v, idx_v, p0_v, p1_v, u_sh, v_sh, sem):
    sid = lax.axis_index("s")
    wid = sid * _NC + lax.axis_index("c")
    base = wid * _EPW
    lo = pl.multiple_of((base // 128) * 128, 128)
    sh = base - lo
    c3a = pltpu.async_copy(eidx_hbm.at[:, pl.ds(lo, _AW2)],
                           idx_v.at[:, pl.ds(0, _AW2)], sem)
    c3b = pltpu.async_copy(eidx_hbm.at[:, pl.ds(lo + _AW2, _AWIN - _AW2)],
                           idx_v.at[:, pl.ds(_AW2, _AWIN - _AW2)], sem)

    @pl.when(sid == 0)
    def _fill():
        pltpu.sync_copy(u_hbm, u_sh)
        pltpu.sync_copy(v_hbm, v_sh)
    plsc.subcore_barrier()
    c1 = pltpu.async_copy(u_sh, u_v, sem)
    c2 = pltpu.async_copy(v_sh, v_v, sem)
    c1.wait()
    c2.wait()
    c3a.wait()

    half = _EPW // 2

    @plsc.parallel_loop(0, half // _L, step=1, unroll=_UNROLL)
    def _loop_a(i):
        off = i * _L
        r = idx_v[0, pl.ds(sh + off, _L)]
        c = idx_v[1, pl.ds(sh + off, _L)]
        us = plsc.load_gather(u_v, [r])
        vs = plsc.load_gather(v_v, [c])
        ed = us * vs
        p0 = 1.0 / (1.0 + ed)
        p0_v[pl.ds(off, _L)] = p0
        p1_v[pl.ds(off, _L)] = 1.0 - p0
    o1 = pltpu.async_copy(p0_v.at[pl.ds(0, half)], out_hbm.at[pl.ds(base, half)], sem)
    o2 = pltpu.async_copy(p1_v.at[pl.ds(0, half)],
                          out_hbm.at[pl.ds(N_EDGES + base, half)], sem)
    c3b.wait()

    @plsc.parallel_loop(half // _L, _EPW // _L, step=1, unroll=_UNROLL)
    def _loop_b(i):
        off = i * _L
        r = idx_v[0, pl.ds(sh + off, _L)]
        c = idx_v[1, pl.ds(sh + off, _L)]
        us = plsc.load_gather(u_v, [r])
        vs = plsc.load_gather(v_v, [c])
        ed = us * vs
        p0 = 1.0 / (1.0 + ed)
        p0_v[pl.ds(off, _L)] = p0
        p1_v[pl.ds(off, _L)] = 1.0 - p0
    o3 = pltpu.async_copy(p0_v.at[pl.ds(half, half)],
                          out_hbm.at[pl.ds(base + half, half)], sem)
    o4 = pltpu.async_copy(p1_v.at[pl.ds(half, half)],
                          out_hbm.at[pl.ds(N_EDGES + base + half, half)], sem)
    o1.wait()
    o2.wait()
    o3.wait()
    o4.wait()


def kernel(node_embeddings, edge_index, W, b):
    u, v = _project(node_embeddings, W, b)
    eidx = edge_index.astype(jnp.int32)
    out_flat = _edge_kernel(u, v, eidx)
    return out_flat.reshape(2, N_EDGES).T
